# baseline (speedup 1.0000x reference)
import functools
import jax
import jax.numpy as jnp
from jax import lax
from jax.experimental import pallas as pl
from jax.experimental.pallas import tpu as pltpu
from jax.experimental.pallas import tpu_sc as plsc

D_MODEL = 768
BATCH = 4
SEQ = 2048
NC, NS = 2, 16
NW = NC * NS
WPB = NW // BATCH
BPW = SEQ // WPB
NSEM = 8

_mesh = plsc.VectorSubcoreMesh(core_axis_name="c", subcore_axis_name="s")


@functools.partial(
    pl.kernel,
    out_type=jax.ShapeDtypeStruct((BATCH, SEQ, D_MODEL), jnp.float32),
    mesh=_mesh,
    scratch_types=[
        pltpu.VMEM((BPW,), jnp.int32),
        pltpu.VMEM_SHARED((NS, BPW), jnp.int32),
        pltpu.SMEM((BPW,), jnp.int32),
        pltpu.SemaphoreType.DMA((NSEM,)),
    ],
)
def _embed_gather(ids_hbm, table_hbm, out_hbm, idx_v, idx_sh, idx_s, sems):
    wid = lax.axis_index("s") * NC + lax.axis_index("c")
    sid = lax.axis_index("s")
    b = wid // WPB
    col0 = (wid % WPB) * BPW
    pltpu.sync_copy(ids_hbm.at[b, pl.ds(col0, BPW)], idx_v)
    pltpu.sync_copy(idx_v, idx_sh.at[sid])
    pltpu.sync_copy(idx_sh.at[sid], idx_s)
    copies = []
    for i in range(BPW):
        idx = idx_s[i]
        copies.append(pltpu.async_copy(
            table_hbm.at[pl.ds(idx, 1)],
            out_hbm.at[b, pl.ds(col0 + i, 1)],
            sems.at[i % NSEM]))
    for c in copies:
        c.wait()


def kernel(input_ids, embed_tokens_weight):
    return _embed_gather(input_ids.astype(jnp.int32), embed_tokens_weight)


# speedup vs baseline: 20.7839x; 20.7839x over previous
import functools

import jax
import jax.numpy as jnp
from jax import lax
from jax.experimental import pallas as pl
from jax.experimental.pallas import tpu as pltpu
from jax.experimental.pallas import tpu_sc as plsc

D_MODEL = 768
BATCH = 4
SEQ = 2048
NC, NS = 2, 16
NW = NC * NS
WPB = NW // BATCH
BPW = SEQ // WPB
CH = 128
NCHUNK = BPW // CH

_mesh = plsc.VectorSubcoreMesh(core_axis_name="c", subcore_axis_name="s")


@functools.partial(
    pl.kernel,
    out_type=jax.ShapeDtypeStruct((BATCH, SEQ, D_MODEL), jnp.float32),
    mesh=_mesh,
    scratch_types=[
        pltpu.VMEM((BPW,), jnp.int32),
        pltpu.VMEM((CH, D_MODEL), jnp.float32),
        pltpu.SemaphoreType.DMA,
        pltpu.SemaphoreType.DMA,
    ],
)
def _embed_gather(ids_hbm, table_hbm, out_hbm, idx_v, rows_v, gsem, ssem):
    wid = lax.axis_index("s") * NC + lax.axis_index("c")
    b = wid // WPB
    col0 = (wid % WPB) * BPW
    pltpu.sync_copy(ids_hbm.at[b, pl.ds(col0, BPW)], idx_v)
    scatter = None
    for c in range(NCHUNK):
        if scatter is not None:
            scatter.wait()
        pltpu.async_copy(
            table_hbm.at[idx_v.at[pl.ds(c * CH, CH)]], rows_v, gsem).wait()
        scatter = pltpu.async_copy(
            rows_v, out_hbm.at[b, pl.ds(col0 + c * CH, CH)], ssem)
    scatter.wait()


def kernel(input_ids, embed_tokens_weight):
    return _embed_gather(input_ids.astype(jnp.int32), embed_tokens_weight)
